# baseline (speedup 1.0000x reference)
import functools

import jax
import jax.numpy as jnp
from jax import lax
from jax.experimental import pallas as pl
from jax.experimental.pallas import tpu as pltpu

B, N, T = 16, 20000, 30
EVAL_NUM = 6
POS_NUM = 10
D = T * 2
BIG = 2**30
NEG = -jnp.inf


def _argmax_rows(work, lin):
    m = jnp.max(work, axis=1)
    cand = jnp.where(work == m[:, None], lin, BIG)
    idx = jnp.min(cand, axis=1)
    oh = lin == idx[:, None]
    return m, idx, oh


def _ext(arr, oh):
    return jnp.sum(jnp.where(oh, arr, 0.0), axis=1)


def _smooth_l1_sum(diff):
    d = jnp.abs(diff)
    return jnp.where(d < 1.0, 0.5 * d * d, d - 0.5)


def _compute_body(x_ref, y_ref, cls_ref, cent_ref, tp_ref, scal_ref, idx_ref):
    x = x_ref[:, :]
    y = y_ref[:, :]
    cls = cls_ref[:, :]
    cent = cent_ref[:, :]
    tx = tp_ref[:, 0]
    ty = tp_ref[:, 1]

    lin = lax.broadcasted_iota(jnp.int32, (B, N), 1)
    bvec = lax.broadcasted_iota(jnp.int32, (B,), 0) * N
    lane16 = lax.broadcasted_iota(jnp.int32, (B, 16), 1)
    lane8 = lax.broadcasted_iota(jnp.int32, (B, 8), 1)

    work = cls
    idxs_acc = jnp.zeros((B, 16), jnp.int32)
    point_sum = jnp.zeros((B,), jnp.float32)
    class_sum = jnp.zeros((B,), jnp.float32)
    cent_sum = jnp.zeros((B,), jnp.float32)
    neg_top_sum = jnp.zeros((B,), jnp.float32)
    for k in range(POS_NUM):
        v, idx, oh = _argmax_rows(work, lin)
        xk = _ext(x, oh)
        yk = _ext(y, oh)
        ck = _ext(cent, oh)
        idxs_acc = idxs_acc + jnp.where(lane16 == k, (idx + bvec)[:, None], 0)
        point_sum = point_sum + _smooth_l1_sum(xk - tx) + _smooth_l1_sum(yk - ty)
        class_sum = class_sum - jnp.log(jnp.clip(v, 1e-7, 1.0 - 1e-7))
        dk = jnp.sqrt((xk - tx) ** 2 + (yk - ty) ** 2 + 1e-12)
        tgt = jnp.where(dk >= 2.0, 0.0, 1.0 - jnp.sqrt(dk / 2.0))
        p = jnp.clip(ck, 1e-7, 1.0 - 1e-7)
        cent_sum = cent_sum - (tgt * jnp.log(p) + (1.0 - tgt) * jnp.log(1.0 - p))
        neg_top_sum = neg_top_sum - jnp.log(jnp.clip(1.0 - v, 1e-7, 1.0))
        work = jnp.where(oh, NEG, work)

    point_loss = point_sum / (POS_NUM * 2)
    class_loss = class_sum / POS_NUM
    centerness_loss = cent_sum / POS_NUM
    neg_all = jnp.sum(-jnp.log(jnp.clip(1.0 - cls, 1e-7, 1.0)), axis=1)
    neg_class_loss = (neg_all - neg_top_sum) / (N - POS_NUM)
    partial_loss = point_loss + class_loss + centerness_loss + neg_class_loss

    comb = cls * cent
    _, _, ohb = _argmax_rows(comb, lin)
    xb = _ext(x, ohb)
    yb = _ext(y, ohb)
    de = jnp.sqrt((xb - tx) ** 2 + (yb - ty) ** 2 + 1e-12)

    scores = comb
    scal = jnp.where(lane8 == 0, partial_loss[:, None], 0.0)
    scal = scal + jnp.where(lane8 == 1, de[:, None], 0.0)
    for k in range(EVAL_NUM):
        _, idx, oh = _argmax_rows(scores, lin)
        pk = _ext(comb, oh)
        xi = _ext(x, oh)
        yi = _ext(y, oh)
        idxs_acc = idxs_acc + jnp.where(
            lane16 == (POS_NUM + k), (idx + bvec)[:, None], 0)
        scal = scal + jnp.where(lane8 == (2 + k), pk[:, None], 0.0)
        d2 = (x - xi[:, None]) ** 2 + (y - yi[:, None]) ** 2
        scores = jnp.where(d2 + 1e-12 < 4.0, NEG, scores)

    scal_ref[:, :] = scal
    idx_ref[:, :] = idxs_acc


def _gather_body(idx_ref, traj_ref, rows_ref, sem):
    copies = []
    for j in range(16):
        c = pltpu.make_async_copy(
            traj_ref.at[idx_ref[0, 0, j]], rows_ref.at[0, j], sem)
        c.start()
        copies.append(c)
    for c in copies:
        c.wait()


def _finish_body(rows_ref, gt_ref, scal_ref, out_ref):
    rows = rows_ref[:, :, :]
    gt = gt_ref[:, :]
    sl1 = _smooth_l1_sum(rows - gt[:, None, :])
    rmask = lax.broadcasted_iota(jnp.int32, (B, 16, 1), 1) < POS_NUM
    traj_loss = jnp.sum(jnp.where(rmask, sl1, 0.0), axis=(1, 2)) / (POS_NUM * D)
    total = scal_ref[:, 0] + traj_loss
    out_ref[:, :] = jnp.reshape(jnp.sum(total) / B, (1, 1))


@jax.jit
def kernel(outputs_coord, outputs_class, outputs_traj, outputs_centerness,
           gt_points):
    interpret = False
    x = outputs_coord[:, :, 0]
    y = outputs_coord[:, :, 1]
    tp = gt_points[:, -1, :]
    traj2 = outputs_traj.reshape(B * N, D)
    gt2 = gt_points.reshape(B, D)

    scal, gidx = pl.pallas_call(
        _compute_body,
        out_shape=(
            jax.ShapeDtypeStruct((B, 8), jnp.float32),
            jax.ShapeDtypeStruct((B, 16), jnp.int32),
        ),
        interpret=interpret,
    )(x, y, outputs_class, outputs_centerness, tp)

    gidx3 = gidx.reshape(B, 1, 16)
    rows = pl.pallas_call(
        _gather_body,
        grid=(B,),
        in_specs=[
            pl.BlockSpec((1, 1, 16), lambda b: (b, 0, 0),
                         memory_space=pltpu.SMEM),
            pl.BlockSpec(memory_space=pl.MemorySpace.ANY),
        ],
        out_specs=pl.BlockSpec((1, 16, D), lambda b: (b, 0, 0)),
        out_shape=jax.ShapeDtypeStruct((B, 16, D), jnp.float32),
        scratch_shapes=[pltpu.SemaphoreType.DMA],
        interpret=interpret,
    )(gidx3, traj2)

    loss = pl.pallas_call(
        _finish_body,
        out_shape=jax.ShapeDtypeStruct((1, 1), jnp.float32),
        interpret=interpret,
    )(rows, gt2, scal)

    return (loss[0, 0], scal[:, 1], rows[:, POS_NUM:, :].reshape(B, EVAL_NUM, T, 2),
            scal[:, 2:8])


# speedup vs baseline: 1.0008x; 1.0008x over previous
import functools

import jax
import jax.numpy as jnp
from jax import lax
from jax.experimental import pallas as pl
from jax.experimental.pallas import tpu as pltpu

B, N, T = 16, 20000, 30
EVAL_NUM = 6
POS_NUM = 10
D = T * 2
BIG = 2**30
NEG = -jnp.inf


def _argmax_rows(work, lin):
    m = jnp.max(work, axis=1)
    cand = jnp.where(work == m[:, None], lin, BIG)
    idx = jnp.min(cand, axis=1)
    oh = lin == idx[:, None]
    return m, idx, oh


def _ext(arr, oh):
    return jnp.sum(jnp.where(oh, arr, 0.0), axis=1)


def _smooth_l1_sum(diff):
    d = jnp.abs(diff)
    return jnp.where(d < 1.0, 0.5 * d * d, d - 0.5)


def _compute_body(x_ref, y_ref, cls_ref, cent_ref, tp_ref, scal_ref, idx_ref):
    x = x_ref[:, :]
    y = y_ref[:, :]
    cls = cls_ref[:, :]
    cent = cent_ref[:, :]
    tx = tp_ref[:, 0]
    ty = tp_ref[:, 1]

    lin = lax.broadcasted_iota(jnp.int32, (B, N), 1)
    bvec = lax.broadcasted_iota(jnp.int32, (B,), 0) * N
    lane16 = lax.broadcasted_iota(jnp.int32, (B, 16), 1)
    lane8 = lax.broadcasted_iota(jnp.int32, (B, 8), 1)

    work = cls
    idxs_acc = jnp.zeros((B, 16), jnp.int32)
    point_sum = jnp.zeros((B,), jnp.float32)
    class_sum = jnp.zeros((B,), jnp.float32)
    cent_sum = jnp.zeros((B,), jnp.float32)
    neg_top_sum = jnp.zeros((B,), jnp.float32)
    for k in range(POS_NUM):
        v, idx, oh = _argmax_rows(work, lin)
        xk = _ext(x, oh)
        yk = _ext(y, oh)
        ck = _ext(cent, oh)
        idxs_acc = idxs_acc + jnp.where(lane16 == k, (idx + bvec)[:, None], 0)
        point_sum = point_sum + _smooth_l1_sum(xk - tx) + _smooth_l1_sum(yk - ty)
        class_sum = class_sum - jnp.log(jnp.clip(v, 1e-7, 1.0 - 1e-7))
        dk = jnp.sqrt((xk - tx) ** 2 + (yk - ty) ** 2 + 1e-12)
        tgt = jnp.where(dk >= 2.0, 0.0, 1.0 - jnp.sqrt(dk / 2.0))
        p = jnp.clip(ck, 1e-7, 1.0 - 1e-7)
        cent_sum = cent_sum - (tgt * jnp.log(p) + (1.0 - tgt) * jnp.log(1.0 - p))
        neg_top_sum = neg_top_sum - jnp.log(jnp.clip(1.0 - v, 1e-7, 1.0))
        work = jnp.where(oh, NEG, work)

    point_loss = point_sum / (POS_NUM * 2)
    class_loss = class_sum / POS_NUM
    centerness_loss = cent_sum / POS_NUM
    neg_all = jnp.sum(-jnp.log(jnp.clip(1.0 - cls, 1e-7, 1.0)), axis=1)
    neg_class_loss = (neg_all - neg_top_sum) / (N - POS_NUM)
    partial_loss = point_loss + class_loss + centerness_loss + neg_class_loss

    comb = cls * cent
    _, _, ohb = _argmax_rows(comb, lin)
    xb = _ext(x, ohb)
    yb = _ext(y, ohb)
    de = jnp.sqrt((xb - tx) ** 2 + (yb - ty) ** 2 + 1e-12)

    scores = comb
    scal = jnp.where(lane8 == 0, partial_loss[:, None], 0.0)
    scal = scal + jnp.where(lane8 == 1, de[:, None], 0.0)
    for k in range(EVAL_NUM):
        _, idx, oh = _argmax_rows(scores, lin)
        pk = _ext(comb, oh)
        xi = _ext(x, oh)
        yi = _ext(y, oh)
        idxs_acc = idxs_acc + jnp.where(
            lane16 == (POS_NUM + k), (idx + bvec)[:, None], 0)
        scal = scal + jnp.where(lane8 == (2 + k), pk[:, None], 0.0)
        d2 = (x - xi[:, None]) ** 2 + (y - yi[:, None]) ** 2
        scores = jnp.where(d2 + 1e-12 < 4.0, NEG, scores)

    scal_ref[:, :] = scal
    idx_ref[:, :] = idxs_acc


def _gather_body(idx_ref, traj_ref, rows_ref, sem):
    copies = []
    for j in range(16):
        c = pltpu.make_async_copy(
            traj_ref.at[idx_ref[0, 0, j]], rows_ref.at[0, j], sem)
        c.start()
        copies.append(c)
    for c in copies:
        c.wait()


def _finish_body(rows_ref, gt_ref, scal_ref, out_ref):
    rows = rows_ref[:, :, :]
    gt = gt_ref[:, :]
    sl1 = _smooth_l1_sum(rows - gt[:, None, :])
    rmask = lax.broadcasted_iota(jnp.int32, (B, 16, 1), 1) < POS_NUM
    traj_loss = jnp.sum(jnp.where(rmask, sl1, 0.0), axis=(1, 2)) / (POS_NUM * D)
    total = scal_ref[:, 0] + traj_loss
    out_ref[:, :] = jnp.reshape(jnp.sum(total) / B, (1, 1))


@jax.jit
def kernel(outputs_coord, outputs_class, outputs_traj, outputs_centerness,
           gt_points):
    interpret = False
    sel = jnp.array([[1.0, 0.0], [0.0, 1.0]], jnp.float32)
    x = jnp.sum(outputs_coord * sel[0], axis=-1)
    y = jnp.sum(outputs_coord * sel[1], axis=-1)
    tp = gt_points[:, -1, :]
    traj2 = outputs_traj.reshape(B * N, D)
    gt2 = gt_points.reshape(B, D)

    scal, gidx = pl.pallas_call(
        _compute_body,
        out_shape=(
            jax.ShapeDtypeStruct((B, 8), jnp.float32),
            jax.ShapeDtypeStruct((B, 16), jnp.int32),
        ),
        interpret=interpret,
    )(x, y, outputs_class, outputs_centerness, tp)

    gidx3 = gidx.reshape(B, 1, 16)
    rows = pl.pallas_call(
        _gather_body,
        grid=(B,),
        in_specs=[
            pl.BlockSpec((1, 1, 16), lambda b: (b, 0, 0),
                         memory_space=pltpu.SMEM),
            pl.BlockSpec(memory_space=pl.MemorySpace.ANY),
        ],
        out_specs=pl.BlockSpec((1, 16, D), lambda b: (b, 0, 0)),
        out_shape=jax.ShapeDtypeStruct((B, 16, D), jnp.float32),
        scratch_shapes=[pltpu.SemaphoreType.DMA],
        interpret=interpret,
    )(gidx3, traj2)

    loss = pl.pallas_call(
        _finish_body,
        out_shape=jax.ShapeDtypeStruct((1, 1), jnp.float32),
        interpret=interpret,
    )(rows, gt2, scal)

    return (loss[0, 0], scal[:, 1], rows[:, POS_NUM:, :].reshape(B, EVAL_NUM, T, 2),
            scal[:, 2:8])
